# baseline (speedup 1.0000x reference)
import jax
import jax.numpy as jnp
from jax.experimental import pallas as pl
from jax.experimental.pallas import tpu as pltpu
from jax.experimental.pallas import tpu_sc as plsc


_CH = 4
_LANES = 16


def _sc_body(x_vmem, pe_vmem, o_vmem):
    ch, b_dim, d = x_vmem.shape

    @pl.loop(0, ch)
    def _(r):
        @pl.loop(0, d, step=4 * _LANES)
        def _(c):
            for u in range(4):
                cc = c + u * _LANES
                pev = pe_vmem.at[r, pl.ds(cc, _LANES)][...]
                for b in range(b_dim):
                    o_vmem.at[r, b, pl.ds(cc, _LANES)][...] = (
                        x_vmem.at[r, b, pl.ds(cc, _LANES)][...] + pev
                    )


def kernel(x, pos_emb):
    S, B, D = x.shape
    pe = pos_emb[:S]
    mesh = plsc.VectorSubcoreMesh(core_axis_name="core",
                                  subcore_axis_name="subcore")

    @pl.kernel(out_type=jax.ShapeDtypeStruct((S, B, D), x.dtype), mesh=mesh)
    def sc_kernel(x_hbm, pe_hbm, o_hbm):
        pltpu.emit_pipeline(
            _sc_body,
            grid=(S // _CH,),
            in_specs=[
                pl.BlockSpec((_CH, B, D), lambda i: (i, 0, 0)),
                pl.BlockSpec((_CH, D), lambda i: (i, 0)),
            ],
            out_specs=[pl.BlockSpec((_CH, B, D), lambda i: (i, 0, 0))],
            core_axis_name=("core", "subcore"),
            dimension_semantics=(pltpu.PARALLEL,),
        )(x_hbm, pe_hbm, o_hbm)

    return sc_kernel(x, pe)


# speedup vs baseline: 1.7733x; 1.7733x over previous
import jax
import jax.numpy as jnp
from jax.experimental import pallas as pl
from jax.experimental.pallas import tpu as pltpu
from jax.experimental.pallas import tpu_sc as plsc


_SC_ROWS = 768
_CH = 4
_LANES = 16
_BS = 256


def _sc_body(x_vmem, pe_vmem, o_vmem):
    ch, b_dim, d = x_vmem.shape

    @pl.loop(0, ch)
    def _(r):
        @pl.loop(0, d, step=4 * _LANES)
        def _(c):
            for u in range(4):
                cc = c + u * _LANES
                pev = pe_vmem.at[r, pl.ds(cc, _LANES)][...]
                for b in range(b_dim):
                    o_vmem.at[r, b, pl.ds(cc, _LANES)][...] = (
                        x_vmem.at[r, b, pl.ds(cc, _LANES)][...] + pev
                    )


def _tc_body(x_ref, pe_ref, o_ref):
    o_ref[...] = x_ref[...] + pe_ref[...][:, None, :]


def kernel(x, pos_emb):
    S, B, D = x.shape
    pe = pos_emb[:S]
    mesh = plsc.VectorSubcoreMesh(core_axis_name="core",
                                  subcore_axis_name="subcore")

    @pl.kernel(out_type=jax.ShapeDtypeStruct((_SC_ROWS, B, D), x.dtype),
               mesh=mesh)
    def sc_kernel(x_hbm, pe_hbm, o_hbm):
        pltpu.emit_pipeline(
            _sc_body,
            grid=(_SC_ROWS // _CH,),
            in_specs=[
                pl.BlockSpec((_CH, B, D), lambda i: (i, 0, 0)),
                pl.BlockSpec((_CH, D), lambda i: (i, 0)),
            ],
            out_specs=[pl.BlockSpec((_CH, B, D), lambda i: (i, 0, 0))],
            core_axis_name=("core", "subcore"),
            dimension_semantics=(pltpu.PARALLEL,),
        )(x_hbm, pe_hbm, o_hbm)

    sc_out = sc_kernel(x, pe)

    tc_rows = S - _SC_ROWS
    off = _SC_ROWS // _BS
    tc_out = pl.pallas_call(
        _tc_body,
        grid=(tc_rows // _BS,),
        in_specs=[
            pl.BlockSpec((_BS, B, D), lambda i: (i + off, 0, 0)),
            pl.BlockSpec((_BS, D), lambda i: (i + off, 0)),
        ],
        out_specs=pl.BlockSpec((_BS, B, D), lambda i: (i, 0, 0)),
        out_shape=jax.ShapeDtypeStruct((tc_rows, B, D), x.dtype),
    )(x, pe)

    return jnp.concatenate([sc_out, tc_out], axis=0)
